# baseline (speedup 1.0000x reference)
import functools

import jax
import jax.numpy as jnp
from jax import lax
from jax.experimental import pallas as pl
from jax.experimental.pallas import tpu as pltpu
from jax.experimental.pallas import tpu_sc as plsc

BATCH = 4096
SEQ = 200
DIM = 64
NTOT = BATCH * SEQ
HALF = BATCH // 2

_info = plsc.get_sparse_core_info()
NC = _info.num_cores
NS = _info.num_subcores
NW = NC * NS
B_PER_W = NTOT // NW
CHUNK = 512
NCHUNK = B_PER_W // CHUNK
NPAIR = NCHUNK // 2

_mesh = plsc.VectorSubcoreMesh(core_axis_name="c", subcore_axis_name="s")


@functools.partial(
    pl.kernel,
    mesh=_mesh,
    out_type=jax.ShapeDtypeStruct((NTOT, DIM), jnp.float32),
    scratch_types=[
        pltpu.VMEM((B_PER_W,), jnp.int32),
        pltpu.VMEM((2, CHUNK, DIM), jnp.float32),
        pltpu.SemaphoreType.DMA,
        pltpu.SemaphoreType.DMA,
        pltpu.SemaphoreType.DMA,
        pltpu.SemaphoreType.DMA,
    ],
    compiler_params=pltpu.CompilerParams(use_tc_tiling_on_sc=False),
)
def _gather_kernel(y_hbm, tab_hbm, out_hbm, idx_v, rows_v, sg0, sg1, sw0, sw1):
    wid = lax.axis_index("s") * NC + lax.axis_index("c")
    base = wid * B_PER_W
    sem_g = (sg0, sg1)
    sem_w = (sw0, sw1)

    pltpu.sync_copy(y_hbm.at[pl.ds(base, B_PER_W)], idx_v)

    def issue_gather(i, b):
        pltpu.async_copy(
            tab_hbm.at[idx_v.at[pl.ds(i * CHUNK, CHUNK)]],
            rows_v.at[b],
            sem_g[b],
        )

    def wait_gather(i, b):
        pltpu.make_async_copy(
            tab_hbm.at[idx_v.at[pl.ds(i * CHUNK, CHUNK)]],
            rows_v.at[b],
            sem_g[b],
        ).wait()

    def issue_wb(i, b):
        pltpu.async_copy(
            rows_v.at[b],
            out_hbm.at[pl.ds(base + i * CHUNK, CHUNK)],
            sem_w[b],
        )

    def wait_wb(b):
        pltpu.make_async_copy(
            rows_v.at[b],
            out_hbm.at[pl.ds(base, CHUNK)],
            sem_w[b],
        ).wait()

    def steady_step(i, b):
        wait_wb(1 - b)
        issue_gather(i + 1, 1 - b)
        wait_gather(i, b)
        issue_wb(i, b)

    issue_gather(0, 0)
    issue_gather(1, 1)
    wait_gather(0, 0)
    issue_wb(0, 0)

    def pair_body(j, carry):
        steady_step(2 * j + 1, 1)
        steady_step(2 * j + 2, 0)
        return carry

    lax.fori_loop(0, NPAIR - 1, pair_body, 0)

    wait_wb(0)
    wait_gather(NCHUNK - 1, 1)
    issue_wb(NCHUNK - 1, 1)
    wait_wb(1)


def _transpose_body(g_ref, out_ref):
    ident = jnp.eye(DIM, dtype=jnp.float32)
    x = g_ref[...]
    for h in range(2):
        part = x[:, h * DIM:(h + 1) * DIM]
        tpart = lax.dot_general(
            ident,
            part,
            (((1,), (1,)), ((), ())),
            precision=lax.Precision.HIGHEST,
            preferred_element_type=jnp.float32,
        )
        out_ref[0, :, h * HALF:(h + 1) * HALF] = tpart


_transpose_call = pl.pallas_call(
    _transpose_body,
    grid=(SEQ,),
    in_specs=[pl.BlockSpec((HALF, 2 * DIM), lambda s: (s, 0))],
    out_specs=pl.BlockSpec((1, DIM, BATCH), lambda s: (s, 0, 0)),
    out_shape=jax.ShapeDtypeStruct((SEQ, DIM, BATCH), jnp.float32),
)


def kernel(y, table):
    y2 = y.reshape(2, HALF, SEQ)
    yf = y2.transpose(2, 1, 0).reshape(NTOT).astype(jnp.int32)
    g = _gather_kernel(yf, table)
    g2 = g.reshape(NTOT // 2, 2 * DIM)
    out3 = _transpose_call(g2)
    return jnp.transpose(out3, (2, 0, 1))


# speedup vs baseline: 1.5549x; 1.5549x over previous
import functools

import jax
import jax.numpy as jnp
from jax import lax
from jax.experimental import pallas as pl
from jax.experimental.pallas import tpu as pltpu
from jax.experimental.pallas import tpu_sc as plsc

BATCH = 4096
SEQ = 200
DIM = 64
NTOT = BATCH * SEQ
HALF = BATCH // 2

_info = plsc.get_sparse_core_info()
NC = _info.num_cores
NS = _info.num_subcores
NW = NC * NS
B_PER_W = NTOT // NW
CHUNK = 512
NCHUNK = B_PER_W // CHUNK
NPAIR = NCHUNK // 2

_mesh = plsc.VectorSubcoreMesh(core_axis_name="c", subcore_axis_name="s")


@functools.partial(
    pl.kernel,
    mesh=_mesh,
    out_type=jax.ShapeDtypeStruct((NTOT, DIM), jnp.float32),
    scratch_types=[
        pltpu.VMEM((B_PER_W,), jnp.int32),
        pltpu.VMEM((2, CHUNK, DIM), jnp.float32),
        pltpu.SemaphoreType.DMA,
        pltpu.SemaphoreType.DMA,
        pltpu.SemaphoreType.DMA,
        pltpu.SemaphoreType.DMA,
    ],
    compiler_params=pltpu.CompilerParams(use_tc_tiling_on_sc=False),
)
def _gather_kernel(y_hbm, tab_hbm, out_hbm, idx_v, rows_v, sg0, sg1, sw0, sw1):
    wid = lax.axis_index("s") * NC + lax.axis_index("c")
    base = wid * B_PER_W
    sem_g = (sg0, sg1)
    sem_w = (sw0, sw1)

    pltpu.sync_copy(y_hbm.at[pl.ds(base, B_PER_W)], idx_v)

    def issue_gather(i, b):
        pltpu.async_copy(
            tab_hbm.at[idx_v.at[pl.ds(i * CHUNK, CHUNK)]],
            rows_v.at[b],
            sem_g[b],
        )

    def wait_gather(i, b):
        pltpu.make_async_copy(
            tab_hbm.at[idx_v.at[pl.ds(i * CHUNK, CHUNK)]],
            rows_v.at[b],
            sem_g[b],
        ).wait()

    def issue_wb(i, b):
        pltpu.async_copy(
            rows_v.at[b],
            out_hbm.at[pl.ds(base + i * CHUNK, CHUNK)],
            sem_w[b],
        )

    def wait_wb(b):
        pltpu.make_async_copy(
            rows_v.at[b],
            out_hbm.at[pl.ds(base, CHUNK)],
            sem_w[b],
        ).wait()

    def steady_step(i, b):
        wait_wb(1 - b)
        issue_gather(i + 1, 1 - b)
        wait_gather(i, b)
        issue_wb(i, b)

    issue_gather(0, 0)
    issue_gather(1, 1)
    wait_gather(0, 0)
    issue_wb(0, 0)

    def pair_body(j, carry):
        steady_step(2 * j + 1, 1)
        steady_step(2 * j + 2, 0)
        return carry

    lax.fori_loop(0, NPAIR - 1, pair_body, 0)

    wait_wb(0)
    wait_gather(NCHUNK - 1, 1)
    issue_wb(NCHUNK - 1, 1)
    wait_wb(1)


SPP = 4


def _transpose_body(g_ref, out_ref):
    xt = g_ref[...].T
    for p in range(SPP):
        for h in range(2):
            out_ref[p, :, h * HALF:(h + 1) * HALF] = xt[
                h * DIM:(h + 1) * DIM, p * HALF:(p + 1) * HALF
            ]


_transpose_call = pl.pallas_call(
    _transpose_body,
    grid=(SEQ // SPP,),
    in_specs=[pl.BlockSpec((SPP * HALF, 2 * DIM), lambda s: (s, 0))],
    out_specs=pl.BlockSpec((SPP, DIM, BATCH), lambda s: (s, 0, 0)),
    out_shape=jax.ShapeDtypeStruct((SEQ, DIM, BATCH), jnp.float32),
)


def kernel(y, table):
    y2 = y.reshape(2, HALF, SEQ)
    yf = y2.transpose(2, 1, 0).reshape(NTOT).astype(jnp.int32)
    g = _gather_kernel(yf, table)
    g2 = g.reshape(NTOT // 2, 2 * DIM)
    out3 = _transpose_call(g2)
    return jnp.transpose(out3, (2, 0, 1))
